# baseline (speedup 1.0000x reference)
import jax
import jax.numpy as jnp
from jax import lax
from jax.experimental import pallas as pl
from jax.experimental.pallas import tpu as pltpu
from jax.experimental.pallas import tpu_sc as plsc

B, K, D, H, W = 8, 4, 384, 32, 32
K2, L = K // 2, H * W
NL = 16
NC, NS = 2, 16
NW = NC * NS
TECS_PER_B = NW // B
D_PER_TEC = D // TECS_PER_B
RD = 8
NBLK = D_PER_TEC // RD
NCH = L // NL


def _argsort_1024(v_ref, occ_ref, bins_ref, asort_ref):
    iota = lax.iota(jnp.int32, NL)

    def zero_body(t, _):
        bins_ref[pl.ds(t * NL, NL)] = jnp.zeros((NL,), jnp.int32)
        return 0
    lax.fori_loop(0, NCH, zero_body, 0)

    def occ_of(base, vv):
        occ = jnp.zeros((NL,), jnp.int32)
        for s in range(1, NL):
            idx = base + jnp.maximum(iota - s, 0)
            shifted = plsc.load_gather(v_ref, [idx])
            hit = jnp.logical_and(vv == shifted, iota >= s)
            occ = occ + hit.astype(jnp.int32)
        return occ

    def hist_body(t, _):
        base = t * NL
        vv = v_ref[pl.ds(base, NL)]
        occ = occ_of(base, vv)
        occ_ref[pl.ds(base, NL)] = occ
        g = plsc.load_gather(bins_ref, [vv])
        plsc.store_scatter(bins_ref, [vv], g + occ + 1)
        return 0
    lax.fori_loop(0, NCH, hist_body, 0)

    def scan_body(t, carry):
        base = t * NL
        c = bins_ref[pl.ds(base, NL)]
        inc = plsc.cumsum(c)
        bins_ref[pl.ds(base, NL)] = inc - c + carry
        return carry + jnp.sum(c)
    lax.fori_loop(0, NCH, scan_body, jnp.int32(0))

    def rank_body(t, _):
        base = t * NL
        vv = v_ref[pl.ds(base, NL)]
        occ = occ_ref[pl.ds(base, NL)]
        g = plsc.load_gather(bins_ref, [vv])
        r = g + occ
        plsc.store_scatter(bins_ref, [vv], r + 1)
        plsc.store_scatter(asort_ref, [r], base + iota)
        return 0
    lax.fori_loop(0, NCH, rank_body, 0)


def _body(ys_hbm, vt_hbm, out_hbm,
          v_ref, occ_ref, bins_ref, as0_ref, as1_ref,
          yb, ob, sem_in, sem_out):
    c = lax.axis_index("c")
    s = lax.axis_index("s")
    wid = s * NC + c
    b = wid // TECS_PER_B
    d_base = (wid % TECS_PER_B) * D_PER_TEC

    def issue_in(blk):
        slot = blk % 2
        d0 = d_base + blk * RD
        return [
            pltpu.async_copy(
                ys_hbm.at[b, k, pl.ds(d0, RD), :], yb.at[slot, k], sem_in.at[slot])
            for k in range(K)
        ]

    desc_in = [issue_in(0), None]
    for k, as_ref in ((0, as0_ref), (1, as1_ref)):
        pltpu.sync_copy(vt_hbm.at[b, k], v_ref)
        _argsort_1024(v_ref, occ_ref, bins_ref, as_ref)

    desc_out = [None, None]
    for blk in range(NBLK):
        slot = blk % 2
        d0 = d_base + blk * RD
        for dsc in desc_in[slot]:
            dsc.wait()
        if blk + 1 < NBLK:
            desc_in[(blk + 1) % 2] = issue_in(blk + 1)
        if desc_out[slot] is not None:
            desc_out[slot].wait()

        def preadd_body(t, _):
            base = t * NL
            rbase = L - NL - base
            for d in range(RD):
                r0 = lax.rev(yb[slot, 2, d, pl.ds(rbase, NL)], (0,))
                yb[slot, 0, d, pl.ds(base, NL)] += r0
                r1 = lax.rev(yb[slot, 3, d, pl.ds(rbase, NL)], (0,))
                yb[slot, 1, d, pl.ds(base, NL)] += r1
            return 0
        lax.fori_loop(0, NCH, preadd_body, 0)

        def chunk_body(t, _):
            base = t * NL
            a0 = as0_ref[pl.ds(base, NL)]
            a1 = as1_ref[pl.ds(base, NL)]
            for d in range(RD):
                dv = jnp.full((NL,), d, jnp.int32)
                acc = (plsc.load_gather(yb.at[slot, 0], [dv, a0])
                       + plsc.load_gather(yb.at[slot, 1], [dv, a1]))
                ob[slot, d, pl.ds(base, NL)] = acc
            return 0
        lax.fori_loop(0, NCH, chunk_body, 0)
        desc_out[slot] = pltpu.async_copy(
            ob.at[slot], out_hbm.at[b, pl.ds(d0, RD), :], sem_out.at[slot])
    for slot in (0, 1):
        if desc_out[slot] is not None:
            desc_out[slot].wait()


def kernel(ys, vec_indices):
    ys4 = ys.reshape(B, K, D, L)
    vt = jnp.transpose(vec_indices, (0, 2, 1))
    mesh = plsc.VectorSubcoreMesh(
        core_axis_name="c", subcore_axis_name="s", num_cores=NC, num_subcores=NS)
    f = pl.kernel(
        _body,
        out_type=jax.ShapeDtypeStruct((B, D, L), jnp.float32),
        mesh=mesh,
        scratch_types=[
            pltpu.VMEM((L,), jnp.int32),
            pltpu.VMEM((L,), jnp.int32),
            pltpu.VMEM((L,), jnp.int32),
            pltpu.VMEM((L,), jnp.int32),
            pltpu.VMEM((L,), jnp.int32),
            pltpu.VMEM((2, K, RD, L), jnp.float32),
            pltpu.VMEM((2, RD, L), jnp.float32),
            pltpu.SemaphoreType.DMA((2,)),
            pltpu.SemaphoreType.DMA((2,)),
        ],
        compiler_params=pltpu.CompilerParams(needs_layout_passes=False),
    )
    return f(ys4, vt)


# speedup vs baseline: 1.9016x; 1.9016x over previous
import jax
import jax.numpy as jnp
from jax import lax
from jax.experimental import pallas as pl
from jax.experimental.pallas import tpu as pltpu
from jax.experimental.pallas import tpu_sc as plsc

B, K, D, H, W = 8, 4, 384, 32, 32
K2, L = K // 2, H * W
NL = 16
NC, NS = 2, 16
NW = NC * NS
TECS_PER_B = NW // B
D_PER_TEC = D // TECS_PER_B
RD = 8
NBLK = D_PER_TEC // RD
NCH = L // NL


def _argsort_1024(v_ref, occ_ref, bins_ref, asort_ref):
    iota = lax.iota(jnp.int32, NL)

    def zero_body(t, _):
        bins_ref[pl.ds(t * NL, NL)] = jnp.zeros((NL,), jnp.int32)
        return 0
    lax.fori_loop(0, NCH, zero_body, 0)

    def occ_of(base, vv):
        occ = jnp.zeros((NL,), jnp.int32)
        for s in range(1, NL):
            idx = base + jnp.maximum(iota - s, 0)
            shifted = plsc.load_gather(v_ref, [idx])
            hit = jnp.logical_and(vv == shifted, iota >= s)
            occ = occ + hit.astype(jnp.int32)
        return occ

    def hist_body(t, _):
        base = t * NL
        vv = v_ref[pl.ds(base, NL)]
        occ = occ_of(base, vv)
        occ_ref[pl.ds(base, NL)] = occ
        g = plsc.load_gather(bins_ref, [vv])
        plsc.store_scatter(bins_ref, [vv], g + occ + 1)
        return 0
    lax.fori_loop(0, NCH, hist_body, 0)

    def scan_body(t, carry):
        base = t * NL
        c = bins_ref[pl.ds(base, NL)]
        inc = plsc.cumsum(c)
        bins_ref[pl.ds(base, NL)] = inc - c + carry
        return carry + jnp.sum(c)
    lax.fori_loop(0, NCH, scan_body, jnp.int32(0))

    def rank_body(t, _):
        base = t * NL
        vv = v_ref[pl.ds(base, NL)]
        occ = occ_ref[pl.ds(base, NL)]
        g = plsc.load_gather(bins_ref, [vv])
        r = g + occ
        plsc.store_scatter(bins_ref, [vv], r + 1)
        plsc.store_scatter(asort_ref, [r], base + iota)
        return 0
    lax.fori_loop(0, NCH, rank_body, 0)


def _body(ys_hbm, vt_hbm, out_hbm,
          v_ref, occ_ref, bins_ref, as0_ref, as1_ref,
          yb, ob, sem_in, sem_out):
    c = lax.axis_index("c")
    s = lax.axis_index("s")
    wid = s * NC + c
    b = wid // TECS_PER_B
    d_base = (wid % TECS_PER_B) * D_PER_TEC

    def issue_in(blk):
        slot = blk % 2
        d0 = d_base + blk * RD
        return [
            pltpu.async_copy(
                ys_hbm.at[b, k, pl.ds(d0, RD), :], yb.at[slot, k], sem_in.at[slot])
            for k in range(K)
        ]

    desc_in = [issue_in(0), None]
    for k, as_ref in ((0, as0_ref), (1, as1_ref)):
        pltpu.sync_copy(vt_hbm.at[b, k], v_ref)
        _argsort_1024(v_ref, occ_ref, bins_ref, as_ref)

    desc_out = [None, None]
    for blk in range(NBLK):
        slot = blk % 2
        d0 = d_base + blk * RD
        for dsc in desc_in[slot]:
            dsc.wait()
        if blk + 1 < NBLK:
            desc_in[(blk + 1) % 2] = issue_in(blk + 1)
        if desc_out[slot] is not None:
            desc_out[slot].wait()

        @plsc.parallel_loop(0, NCH, 1, unroll=2)
        def chunk_body(t):
            base = t * NL
            a0 = as0_ref[pl.ds(base, NL)]
            a1 = as1_ref[pl.ds(base, NL)]
            f0 = (L - 1) - a0
            f1 = (L - 1) - a1
            for d in range(RD):
                dv = jnp.full((NL,), d, jnp.int32)
                acc = (plsc.load_gather(yb.at[slot, 0], [dv, a0])
                       + plsc.load_gather(yb.at[slot, 2], [dv, f0])
                       + plsc.load_gather(yb.at[slot, 1], [dv, a1])
                       + plsc.load_gather(yb.at[slot, 3], [dv, f1]))
                ob[slot, d, pl.ds(base, NL)] = acc
        desc_out[slot] = pltpu.async_copy(
            ob.at[slot], out_hbm.at[b, pl.ds(d0, RD), :], sem_out.at[slot])
    for slot in (0, 1):
        if desc_out[slot] is not None:
            desc_out[slot].wait()


def kernel(ys, vec_indices):
    ys4 = ys.reshape(B, K, D, L)
    vt = jnp.transpose(vec_indices, (0, 2, 1))
    mesh = plsc.VectorSubcoreMesh(
        core_axis_name="c", subcore_axis_name="s", num_cores=NC, num_subcores=NS)
    f = pl.kernel(
        _body,
        out_type=jax.ShapeDtypeStruct((B, D, L), jnp.float32),
        mesh=mesh,
        scratch_types=[
            pltpu.VMEM((L,), jnp.int32),
            pltpu.VMEM((L,), jnp.int32),
            pltpu.VMEM((L,), jnp.int32),
            pltpu.VMEM((L,), jnp.int32),
            pltpu.VMEM((L,), jnp.int32),
            pltpu.VMEM((2, K, RD, L), jnp.float32),
            pltpu.VMEM((2, RD, L), jnp.float32),
            pltpu.SemaphoreType.DMA((2,)),
            pltpu.SemaphoreType.DMA((2,)),
        ],
        compiler_params=pltpu.CompilerParams(needs_layout_passes=False),
    )
    return f(ys4, vt)
